# baseline (speedup 1.0000x reference)
import functools

import jax
import jax.numpy as jnp
from jax.experimental import pallas as pl
from jax.experimental.pallas import tpu as pltpu

B, C, H, W = 4, 384, 224, 224
HW = H * W
K = 96
HID = C // 2
CB = 32
NCB = C // CB

_I32_MIN = jnp.iinfo(jnp.int32).min
_I32_MAX = jnp.iinfo(jnp.int32).max


def _stats_body(x_ref, o_ref):
    xb = x_ref[0].reshape(CB, HW)
    n = jnp.float32(HW)
    mean = jnp.sum(xb, axis=1) / n
    var = jnp.sum((xb - mean[:, None]) ** 2, axis=1) / n
    std = jnp.sqrt(var)
    mx = jnp.max(xb, axis=1)

    bits = jax.lax.bitcast_convert_type(xb, jnp.int32)
    key = bits ^ ((bits >> 31) & jnp.int32(0x7FFFFFFF))

    target = HW // 2

    def body(_, carry):
        lo, hi = carry
        mid = lo + (((hi - lo) >> 1) & jnp.int32(0x7FFFFFFF))
        cnt = jnp.sum((key <= mid).astype(jnp.int32), axis=1,
                      keepdims=True)
        pred = cnt >= target
        lo = jnp.where(pred, lo, mid + 1)
        hi = jnp.where(pred, mid, hi)
        return lo, hi

    lo0 = jnp.full((CB, 1), _I32_MIN, jnp.int32)
    hi0 = jnp.full((CB, 1), _I32_MAX, jnp.int32)
    m1, _ = jax.lax.fori_loop(0, 32, body, (lo0, hi0))

    cnt_le = jnp.sum((key <= m1).astype(jnp.int32), axis=1, keepdims=True)
    above = jnp.where(key > m1, key, _I32_MAX)
    m2 = jnp.where(cnt_le >= target + 1, m1,
                   jnp.min(above, axis=1, keepdims=True))

    def unkey(k):
        b = jnp.where(k < 0, k ^ jnp.int32(0x7FFFFFFF), k)
        return jax.lax.bitcast_convert_type(b, jnp.float32)

    med = 0.5 * (unkey(m1) + unkey(m2))[:, 0]
    o_ref[...] = jnp.stack([std, med, mx])[None, None]


def _topk_body(std_ref, med_ref, max_ref, w1_ref, b1_ref, w2_ref, b2_ref,
               idx_ref):
    w1 = w1_ref[...]
    w2 = w2_ref[...]
    b1 = b1_ref[...]
    b2 = b2_ref[...]
    logits = jnp.zeros((B, C), jnp.float32)
    for s_ref in (std_ref, med_ref, max_ref):
        s = s_ref[...]
        hid = jnp.maximum(
            jnp.dot(s, w1, preferred_element_type=jnp.float32) + b1, 0.0)
        logits += jnp.dot(hid, w2, preferred_element_type=jnp.float32) + b2
    logits = logits / 3.0

    ii = jax.lax.broadcasted_iota(jnp.int32, (C, C), 0)
    jj = jax.lax.broadcasted_iota(jnp.int32, (C, C), 1)
    kk = jax.lax.broadcasted_iota(jnp.int32, (C, 128), 1)
    ir = jax.lax.broadcasted_iota(jnp.int32, (C, 128), 0)
    for b in range(B):
        lrow = logits[b:b + 1, :]
        lmat = jnp.broadcast_to(lrow, (C, C))
        lcol = jnp.sum(jnp.where(ii == jj, lmat, 0.0), axis=1,
                       keepdims=True)
        beats = (lmat > lcol) | ((lmat == lcol) & (jj < ii))
        rank = jnp.sum(beats.astype(jnp.int32), axis=1, keepdims=True)
        sel = (rank == kk)
        idx_ref[b:b + 1, :] = jnp.sum(jnp.where(sel, ir, 0), axis=0,
                                      keepdims=True)


def _gather_body(idx_ref, x_ref, o_ref):
    del idx_ref
    o_ref[...] = x_ref[...]


@functools.partial(jax.jit)
def kernel(x, W1, b1, W2, b2):
    stats = pl.pallas_call(
        _stats_body,
        grid=(B, NCB),
        in_specs=[pl.BlockSpec((1, CB, H, W), lambda b, j: (b, j, 0, 0))],
        out_specs=pl.BlockSpec((1, 1, 3, CB), lambda b, j: (b, j, 0, 0)),
        out_shape=jax.ShapeDtypeStruct((B, NCB, 3, CB), jnp.float32),
        compiler_params=pltpu.CompilerParams(
            dimension_semantics=("parallel", "parallel")),
    )(x)
    stats = stats.transpose(2, 0, 1, 3).reshape(3, B, C)

    idx = pl.pallas_call(
        _topk_body,
        out_shape=jax.ShapeDtypeStruct((B, 128), jnp.int32),
    )(stats[0], stats[1], stats[2], W1, b1.reshape(1, HID), W2,
      b2.reshape(1, C))
    idx = idx[:, :K]

    out = pl.pallas_call(
        _gather_body,
        grid_spec=pltpu.PrefetchScalarGridSpec(
            num_scalar_prefetch=1,
            grid=(B, K),
            in_specs=[pl.BlockSpec(
                (1, 1, H, W),
                lambda b, k, idx_ref: (b, idx_ref[b, k], 0, 0))],
            out_specs=pl.BlockSpec(
                (1, 1, H, W), lambda b, k, idx_ref: (b, k, 0, 0)),
        ),
        out_shape=jax.ShapeDtypeStruct((B, K, H, W), jnp.float32),
    )(idx, x)
    return out


# speedup vs baseline: 1.1398x; 1.1398x over previous
import functools

import jax
import jax.numpy as jnp
from jax.experimental import pallas as pl
from jax.experimental.pallas import tpu as pltpu

B, C, H, W = 4, 384, 224, 224
HW = H * W
K = 96
HID = C // 2
CB = 64
NCB = C // CB

_I32_MIN = jnp.iinfo(jnp.int32).min
_I32_MAX = jnp.iinfo(jnp.int32).max


def _stats_body(x_ref, o_ref):
    xb = x_ref[0].reshape(CB, HW)
    n = jnp.float32(HW)
    mean = jnp.sum(xb, axis=1) / n
    var = jnp.sum((xb - mean[:, None]) ** 2, axis=1) / n
    std = jnp.sqrt(var)
    mx = jnp.max(xb, axis=1)

    bits = jax.lax.bitcast_convert_type(xb, jnp.int32)
    key = bits ^ ((bits >> 31) & jnp.int32(0x7FFFFFFF))

    target = HW // 2

    def body(_, carry):
        lo, hi = carry
        mid = lo + (((hi - lo) >> 1) & jnp.int32(0x7FFFFFFF))
        cnt = jnp.sum((key <= mid).astype(jnp.int32), axis=1,
                      keepdims=True)
        pred = cnt >= target
        lo = jnp.where(pred, lo, mid + 1)
        hi = jnp.where(pred, mid, hi)
        return lo, hi

    lo0 = jnp.full((CB, 1), _I32_MIN, jnp.int32)
    hi0 = jnp.full((CB, 1), _I32_MAX, jnp.int32)
    m1, _ = jax.lax.fori_loop(0, 32, body, (lo0, hi0))

    cnt_le = jnp.sum((key <= m1).astype(jnp.int32), axis=1, keepdims=True)
    above = jnp.where(key > m1, key, _I32_MAX)
    m2 = jnp.where(cnt_le >= target + 1, m1,
                   jnp.min(above, axis=1, keepdims=True))

    def unkey(k):
        b = jnp.where(k < 0, k ^ jnp.int32(0x7FFFFFFF), k)
        return jax.lax.bitcast_convert_type(b, jnp.float32)

    med = 0.5 * (unkey(m1) + unkey(m2))[:, 0]
    o_ref[...] = jnp.stack([std, med, mx])[None, None]


def _topk_body(std_ref, med_ref, max_ref, w1_ref, b1_ref, w2_ref, b2_ref,
               idx_ref):
    w1 = w1_ref[...]
    w2 = w2_ref[...]
    b1 = b1_ref[...]
    b2 = b2_ref[...]
    logits = jnp.zeros((B, C), jnp.float32)
    for s_ref in (std_ref, med_ref, max_ref):
        s = s_ref[...]
        hid = jnp.maximum(
            jnp.dot(s, w1, preferred_element_type=jnp.float32) + b1, 0.0)
        logits += jnp.dot(hid, w2, preferred_element_type=jnp.float32) + b2
    logits = logits / 3.0

    ii = jax.lax.broadcasted_iota(jnp.int32, (C, C), 0)
    jj = jax.lax.broadcasted_iota(jnp.int32, (C, C), 1)
    kk = jax.lax.broadcasted_iota(jnp.int32, (C, 128), 1)
    ir = jax.lax.broadcasted_iota(jnp.int32, (C, 128), 0)
    for b in range(B):
        lrow = logits[b:b + 1, :]
        lmat = jnp.broadcast_to(lrow, (C, C))
        lcol = jnp.sum(jnp.where(ii == jj, lmat, 0.0), axis=1,
                       keepdims=True)
        beats = (lmat > lcol) | ((lmat == lcol) & (jj < ii))
        rank = jnp.sum(beats.astype(jnp.int32), axis=1, keepdims=True)
        sel = (rank == kk)
        idx_ref[b:b + 1, :] = jnp.sum(jnp.where(sel, ir, 0), axis=0,
                                      keepdims=True)


def _gather_body(idx_ref, x_ref, o_ref):
    del idx_ref
    o_ref[...] = x_ref[...]


@functools.partial(jax.jit)
def kernel(x, W1, b1, W2, b2):
    stats = pl.pallas_call(
        _stats_body,
        grid=(B, NCB),
        in_specs=[pl.BlockSpec((1, CB, H, W), lambda b, j: (b, j, 0, 0))],
        out_specs=pl.BlockSpec((1, 1, 3, CB), lambda b, j: (b, j, 0, 0)),
        out_shape=jax.ShapeDtypeStruct((B, NCB, 3, CB), jnp.float32),
        compiler_params=pltpu.CompilerParams(
            dimension_semantics=("parallel", "parallel")),
    )(x)
    stats = stats.transpose(2, 0, 1, 3).reshape(3, B, C)

    idx = pl.pallas_call(
        _topk_body,
        out_shape=jax.ShapeDtypeStruct((B, 128), jnp.int32),
    )(stats[0], stats[1], stats[2], W1, b1.reshape(1, HID), W2,
      b2.reshape(1, C))
    idx = idx[:, :K]

    out = pl.pallas_call(
        _gather_body,
        grid_spec=pltpu.PrefetchScalarGridSpec(
            num_scalar_prefetch=1,
            grid=(B, K),
            in_specs=[pl.BlockSpec(
                (1, 1, H, W),
                lambda b, k, idx_ref: (b, idx_ref[b, k], 0, 0))],
            out_specs=pl.BlockSpec(
                (1, 1, H, W), lambda b, k, idx_ref: (b, k, 0, 0)),
        ),
        out_shape=jax.ShapeDtypeStruct((B, K, H, W), jnp.float32),
    )(idx, x)
    return out
